# Initial kernel scaffold; baseline (speedup 1.0000x reference)
import jax
import jax.numpy as jnp
from jax.experimental import pallas as pl


def kernel(x, edge_index, W1, b1, W2, b2, W3, b3):
    raise NotImplementedError("write your pallas kernel here")


# speedup vs baseline: 4.9941x; 4.9941x over previous
import functools

import jax
import jax.numpy as jnp
from jax import lax
from jax.experimental import pallas as pl
from jax.experimental.pallas import tpu as pltpu
from jax.experimental.pallas import tpu_sc as plsc

_N = 10000
_E = 160000
_H = 512
_NCH = 4
_CD = 128
_NCORE = 2
_NSUB = 16
_NP = 10240
_RT = _NP // _NSUB
_EP = 163840
_BE = 128
_ET = _EP // _NSUB
_J = _ET // _BE
_NB = 1000
_G = _N // _NB


def _mesh():
    return plsc.VectorSubcoreMesh(core_axis_name="c", subcore_axis_name="s",
                                  num_cores=_NCORE, num_subcores=_NSUB)


def _sc_deg_body(edges_hbm, zeros_hbm, ones_hbm, out_hbm, cbuf, val, deg_sh):
    core = lax.axis_index("c")
    sub = lax.axis_index("s")
    r0 = sub * _RT
    pltpu.sync_copy(zeros_hbm, deg_sh.at[pl.ds(r0, _RT)])
    pltpu.sync_copy(ones_hbm, val)
    plsc.subcore_barrier()

    def body(j, c):
        g = sub * _J + j
        pltpu.sync_copy(edges_hbm.at[g], cbuf.at[0])
        pltpu.sync_copy(val, deg_sh.at[cbuf.at[0].at[1]], add=True)
        return c

    lax.fori_loop(0, _J, body, 0)
    plsc.subcore_barrier()

    @pl.when(core == 0)
    def _():
        pltpu.sync_copy(deg_sh.at[pl.ds(r0, _RT)], out_hbm.at[pl.ds(r0, _RT)])


@functools.lru_cache(maxsize=None)
def _sc_deg():
    return pl.kernel(
        _sc_deg_body,
        out_type=jax.ShapeDtypeStruct((_NP, _CD), jnp.float32),
        mesh=_mesh(),
        scratch_types=[
            pltpu.VMEM((1, 2, _BE), jnp.int32),
            pltpu.VMEM((_BE, _CD), jnp.float32),
            pltpu.VMEM_SHARED((_NP, _CD), jnp.float32),
        ],
    )


def _sc_agg_body(t_hbm, edges_hbm, zeros_hbm, out_hbm, ebuf, rows, acc_sh,
                 gsem0, gsem1):
    core = lax.axis_index("c")
    sub = lax.axis_index("s")
    r0 = sub * _RT
    sems = (gsem0, gsem1)
    for p in range(2):
        ch = core * 2 + p
        pltpu.sync_copy(zeros_hbm, acc_sh.at[pl.ds(r0, _RT)])
        plsc.subcore_barrier()

        def body(jo, c):
            for q in range(2):
                g = sub * _J + jo * 2 + q
                pltpu.sync_copy(edges_hbm.at[g], ebuf.at[q])
                pltpu.async_copy(t_hbm.at[ch].at[ebuf.at[q].at[0]],
                                 rows.at[q], sems[q])
            for q in range(2):
                pltpu.make_async_copy(t_hbm.at[ch].at[ebuf.at[q].at[0]],
                                      rows.at[q], sems[q]).wait()
                pltpu.sync_copy(rows.at[q], acc_sh.at[ebuf.at[q].at[1]],
                                add=True)
            return c

        lax.fori_loop(0, _J // 2, body, 0)
        plsc.subcore_barrier()
        pltpu.sync_copy(acc_sh.at[pl.ds(r0, _RT)],
                        out_hbm.at[ch].at[pl.ds(r0, _RT)])
        plsc.subcore_barrier()


@functools.lru_cache(maxsize=None)
def _sc_agg():
    return pl.kernel(
        _sc_agg_body,
        out_type=jax.ShapeDtypeStruct((_NCH, _NP, _CD), jnp.float32),
        mesh=_mesh(),
        scratch_types=[
            pltpu.VMEM((2, 2, _BE), jnp.int32),
            pltpu.VMEM((2, _BE, _CD), jnp.float32),
            pltpu.VMEM_SHARED((_NP, _CD), jnp.float32),
            pltpu.SemaphoreType.DMA,
            pltpu.SemaphoreType.DMA,
        ],
    )


def _mm_first_body(x_ref, w_ref, deg_ref, o_ref):
    dis = lax.rsqrt(deg_ref[...] + 1.0)
    y = jnp.dot(x_ref[...], w_ref[...], preferred_element_type=jnp.float32,
                precision=lax.Precision.HIGHEST)
    t = y * dis
    for c in range(_NCH):
        o_ref[c] = t[:, c * _CD:(c + 1) * _CD]


def _mm_first(x, w, deg2):
    return pl.pallas_call(
        _mm_first_body,
        grid=(_G,),
        in_specs=[
            pl.BlockSpec((_NB, x.shape[1]), lambda i: (i, 0)),
            pl.BlockSpec((w.shape[0], _H), lambda i: (0, 0)),
            pl.BlockSpec((_NB, 1), lambda i: (i, 0)),
        ],
        out_specs=pl.BlockSpec((_NCH, _NB, _CD), lambda i: (0, i, 0)),
        out_shape=jax.ShapeDtypeStruct((_NCH, _N, _CD), jnp.float32),
    )(x, w, deg2)


def _mm_mid_body(a_ref, t_ref, deg_ref, b_ref, w_ref, o_ref):
    dis = lax.rsqrt(deg_ref[...] + 1.0)
    s = jnp.concatenate([a_ref[c] + t_ref[c] for c in range(_NCH)], axis=1)
    h = s * dis + b_ref[...]
    h = jnp.where(h > 0, h, jnp.exp(jnp.minimum(h, 0.0)) - 1.0)
    y = jnp.dot(h, w_ref[...], preferred_element_type=jnp.float32,
                precision=lax.Precision.HIGHEST)
    t = y * dis
    for c in range(_NCH):
        o_ref[c] = t[:, c * _CD:(c + 1) * _CD]


def _mm_mid(a, t, deg2, b2d, w):
    return pl.pallas_call(
        _mm_mid_body,
        grid=(_G,),
        in_specs=[
            pl.BlockSpec((_NCH, _NB, _CD), lambda i: (0, i, 0)),
            pl.BlockSpec((_NCH, _NB, _CD), lambda i: (0, i, 0)),
            pl.BlockSpec((_NB, 1), lambda i: (i, 0)),
            pl.BlockSpec((1, _H), lambda i: (0, 0)),
            pl.BlockSpec((_H, _H), lambda i: (0, 0)),
        ],
        out_specs=pl.BlockSpec((_NCH, _NB, _CD), lambda i: (0, i, 0)),
        out_shape=jax.ShapeDtypeStruct((_NCH, _N, _CD), jnp.float32),
    )(a, t, deg2, b2d, w)


def _final_body(a_ref, t_ref, deg_ref, b_ref, o_ref):
    dis = lax.rsqrt(deg_ref[...] + 1.0)
    s = jnp.concatenate([a_ref[c] + t_ref[c] for c in range(_NCH)], axis=1)
    o_ref[...] = s * dis + b_ref[...]


def _final(a, t, deg2, b2d):
    return pl.pallas_call(
        _final_body,
        grid=(_G,),
        in_specs=[
            pl.BlockSpec((_NCH, _NB, _CD), lambda i: (0, i, 0)),
            pl.BlockSpec((_NCH, _NB, _CD), lambda i: (0, i, 0)),
            pl.BlockSpec((_NB, 1), lambda i: (i, 0)),
            pl.BlockSpec((1, _H), lambda i: (0, 0)),
        ],
        out_specs=pl.BlockSpec((_NB, _H), lambda i: (i, 0)),
        out_shape=jax.ShapeDtypeStruct((_N, _H), jnp.float32),
    )(a, t, deg2, b2d)


def kernel(x, edge_index, W1, b1, W2, b2, W3, b3):
    row = edge_index[0].astype(jnp.int32)
    col = edge_index[1].astype(jnp.int32)
    pad = _EP - _E
    rowp = jnp.concatenate([row, jnp.zeros((pad,), jnp.int32)])
    colp = jnp.concatenate([col, jnp.full((pad,), _N, jnp.int32)])
    edges = jnp.stack([rowp.reshape(_EP // _BE, _BE),
                       colp.reshape(_EP // _BE, _BE)], axis=1)
    zeros_e = jnp.zeros((_RT, _CD), jnp.float32)

    ones_d = jnp.zeros((_BE, _CD), jnp.float32).at[:, 0].set(1.0)
    degt = _sc_deg()(edges, zeros_e, ones_d)
    deg2 = degt[:_N, 0:1]

    t1 = _mm_first(x, W1, deg2)
    a1 = _sc_agg()(t1, edges, zeros_e)[:, :_N, :]
    t2 = _mm_mid(a1, t1, deg2, b1.reshape(1, _H), W2)
    a2 = _sc_agg()(t2, edges, zeros_e)[:, :_N, :]
    t3 = _mm_mid(a2, t2, deg2, b2.reshape(1, _H), W3)
    a3 = _sc_agg()(t3, edges, zeros_e)[:, :_N, :]
    return _final(a3, t3, deg2, b3.reshape(1, _H))


# speedup vs baseline: 5.0310x; 1.0074x over previous
import functools

import jax
import jax.numpy as jnp
from jax import lax
from jax.experimental import pallas as pl
from jax.experimental.pallas import tpu as pltpu
from jax.experimental.pallas import tpu_sc as plsc

_N = 10000
_E = 160000
_H = 512
_NCH = 4
_CD = 128
_NCORE = 2
_NSUB = 16
_NP = 10240
_RT = _NP // _NSUB
_EP = 163840
_BE = 128
_ET = _EP // _NSUB
_J = _ET // _BE
_NB = 1000
_G = _N // _NB


def _mesh():
    return plsc.VectorSubcoreMesh(core_axis_name="c", subcore_axis_name="s",
                                  num_cores=_NCORE, num_subcores=_NSUB)


def _sc_deg_body(edges_hbm, zeros_hbm, ones_hbm, out_hbm, cbuf, val, deg_sh):
    core = lax.axis_index("c")
    sub = lax.axis_index("s")
    r0 = sub * _RT
    pltpu.sync_copy(zeros_hbm, deg_sh.at[pl.ds(r0, _RT)])
    pltpu.sync_copy(ones_hbm, val)
    plsc.subcore_barrier()

    def body(j, c):
        g = sub * _J + j
        pltpu.sync_copy(edges_hbm.at[g], cbuf.at[0])
        pltpu.sync_copy(val, deg_sh.at[cbuf.at[0].at[1]], add=True)
        return c

    lax.fori_loop(0, _J, body, 0)
    plsc.subcore_barrier()

    @pl.when(core == 0)
    def _():
        pltpu.sync_copy(deg_sh.at[pl.ds(r0, _RT)], out_hbm.at[pl.ds(r0, _RT)])


@functools.lru_cache(maxsize=None)
def _sc_deg():
    return pl.kernel(
        _sc_deg_body,
        out_type=jax.ShapeDtypeStruct((_NP, _CD), jnp.float32),
        mesh=_mesh(),
        scratch_types=[
            pltpu.VMEM((1, 2, _BE), jnp.int32),
            pltpu.VMEM((_BE, _CD), jnp.float32),
            pltpu.VMEM_SHARED((_NP, _CD), jnp.float32),
        ],
    )


_NS = 2
_NG = _J // _NS


def _sc_agg_body(t_hbm, edges_hbm, zeros_hbm, out_hbm, ibuf, rows, acc_sh,
                 *sems):
    core = lax.axis_index("c")
    sub = lax.axis_index("s")
    r0 = sub * _RT
    isems = sems[0:2]
    gsems = sems[2:2 + _NS]
    ssems = sems[4:4 + _NS]
    e0 = sub * _J

    for p in range(2):
        ch = core * 2 + p
        pltpu.sync_copy(zeros_hbm, acc_sh.at[pl.ds(r0, _RT)])
        plsc.subcore_barrier()
        pltpu.async_copy(edges_hbm.at[pl.ds(e0, _NS)], ibuf.at[0], isems[0])

        def sgroup(g2, c):
            for par in range(2):
                g = g2 * 2 + par
                pltpu.make_async_copy(edges_hbm.at[pl.ds(e0 + g * _NS, _NS)],
                                      ibuf.at[par], isems[par]).wait()

                def retire():
                    for k in range(_NS):
                        pltpu.make_async_copy(
                            rows.at[k], acc_sh.at[ibuf.at[par].at[k].at[1]],
                            ssems[k]).wait()

                if par == 0:
                    pl.when(g2 > 0)(retire)
                else:
                    retire()

                @pl.when(g < _NG - 1)
                def _():
                    pltpu.async_copy(
                        edges_hbm.at[pl.ds(e0 + (g + 1) * _NS, _NS)],
                        ibuf.at[1 - par], isems[1 - par])

                for k in range(_NS):
                    pltpu.async_copy(t_hbm.at[ch].at[ibuf.at[par].at[k].at[0]],
                                     rows.at[k], gsems[k])
                for k in range(_NS):
                    pltpu.make_async_copy(
                        t_hbm.at[ch].at[ibuf.at[par].at[k].at[0]],
                        rows.at[k], gsems[k]).wait()
                    pltpu.async_copy(rows.at[k],
                                     acc_sh.at[ibuf.at[par].at[k].at[1]],
                                     ssems[k], add=True)
            return c

        lax.fori_loop(0, _NG // 2, sgroup, 0)
        for k in range(_NS):
            pltpu.make_async_copy(rows.at[k],
                                  acc_sh.at[ibuf.at[1].at[k].at[1]],
                                  ssems[k]).wait()
        plsc.subcore_barrier()
        pltpu.sync_copy(acc_sh.at[pl.ds(r0, _RT)],
                        out_hbm.at[ch].at[pl.ds(r0, _RT)])
        plsc.subcore_barrier()


@functools.lru_cache(maxsize=None)
def _sc_agg():
    return pl.kernel(
        _sc_agg_body,
        out_type=jax.ShapeDtypeStruct((_NCH, _NP, _CD), jnp.float32),
        mesh=_mesh(),
        scratch_types=[
            pltpu.VMEM((2, _NS, 2, _BE), jnp.int32),
            pltpu.VMEM((_NS, _BE, _CD), jnp.float32),
            pltpu.VMEM_SHARED((_NP, _CD), jnp.float32),
        ] + [pltpu.SemaphoreType.DMA] * 6,
    )


def _mm_first_body(x_ref, w_ref, deg_ref, o_ref):
    dis = lax.rsqrt(deg_ref[...] + 1.0)
    y = jnp.dot(x_ref[...], w_ref[...], preferred_element_type=jnp.float32,
                precision=lax.Precision.HIGHEST)
    t = y * dis
    for c in range(_NCH):
        o_ref[c] = t[:, c * _CD:(c + 1) * _CD]


def _mm_first(x, w, deg2):
    return pl.pallas_call(
        _mm_first_body,
        grid=(_G,),
        in_specs=[
            pl.BlockSpec((_NB, x.shape[1]), lambda i: (i, 0)),
            pl.BlockSpec((w.shape[0], _H), lambda i: (0, 0)),
            pl.BlockSpec((_NB, 1), lambda i: (i, 0)),
        ],
        out_specs=pl.BlockSpec((_NCH, _NB, _CD), lambda i: (0, i, 0)),
        out_shape=jax.ShapeDtypeStruct((_NCH, _N, _CD), jnp.float32),
    )(x, w, deg2)


def _mm_mid_body(a_ref, t_ref, deg_ref, b_ref, w_ref, o_ref):
    dis = lax.rsqrt(deg_ref[...] + 1.0)
    s = jnp.concatenate([a_ref[c] + t_ref[c] for c in range(_NCH)], axis=1)
    h = s * dis + b_ref[...]
    h = jnp.where(h > 0, h, jnp.exp(jnp.minimum(h, 0.0)) - 1.0)
    y = jnp.dot(h, w_ref[...], preferred_element_type=jnp.float32,
                precision=lax.Precision.HIGHEST)
    t = y * dis
    for c in range(_NCH):
        o_ref[c] = t[:, c * _CD:(c + 1) * _CD]


def _mm_mid(a, t, deg2, b2d, w):
    return pl.pallas_call(
        _mm_mid_body,
        grid=(_G,),
        in_specs=[
            pl.BlockSpec((_NCH, _NB, _CD), lambda i: (0, i, 0)),
            pl.BlockSpec((_NCH, _NB, _CD), lambda i: (0, i, 0)),
            pl.BlockSpec((_NB, 1), lambda i: (i, 0)),
            pl.BlockSpec((1, _H), lambda i: (0, 0)),
            pl.BlockSpec((_H, _H), lambda i: (0, 0)),
        ],
        out_specs=pl.BlockSpec((_NCH, _NB, _CD), lambda i: (0, i, 0)),
        out_shape=jax.ShapeDtypeStruct((_NCH, _N, _CD), jnp.float32),
    )(a, t, deg2, b2d, w)


def _final_body(a_ref, t_ref, deg_ref, b_ref, o_ref):
    dis = lax.rsqrt(deg_ref[...] + 1.0)
    s = jnp.concatenate([a_ref[c] + t_ref[c] for c in range(_NCH)], axis=1)
    o_ref[...] = s * dis + b_ref[...]


def _final(a, t, deg2, b2d):
    return pl.pallas_call(
        _final_body,
        grid=(_G,),
        in_specs=[
            pl.BlockSpec((_NCH, _NB, _CD), lambda i: (0, i, 0)),
            pl.BlockSpec((_NCH, _NB, _CD), lambda i: (0, i, 0)),
            pl.BlockSpec((_NB, 1), lambda i: (i, 0)),
            pl.BlockSpec((1, _H), lambda i: (0, 0)),
        ],
        out_specs=pl.BlockSpec((_NB, _H), lambda i: (i, 0)),
        out_shape=jax.ShapeDtypeStruct((_N, _H), jnp.float32),
    )(a, t, deg2, b2d)


def kernel(x, edge_index, W1, b1, W2, b2, W3, b3):
    row = edge_index[0].astype(jnp.int32)
    col = edge_index[1].astype(jnp.int32)
    pad = _EP - _E
    rowp = jnp.concatenate([row, jnp.zeros((pad,), jnp.int32)])
    colp = jnp.concatenate([col, jnp.full((pad,), _N, jnp.int32)])
    edges = jnp.stack([rowp.reshape(_EP // _BE, _BE),
                       colp.reshape(_EP // _BE, _BE)], axis=1)
    zeros_e = jnp.zeros((_RT, _CD), jnp.float32)

    ones_d = jnp.zeros((_BE, _CD), jnp.float32).at[:, 0].set(1.0)
    degt = _sc_deg()(edges, zeros_e, ones_d)
    deg2 = degt[:_N, 0:1]

    t1 = _mm_first(x, W1, deg2)
    a1 = _sc_agg()(t1, edges, zeros_e)[:, :_N, :]
    t2 = _mm_mid(a1, t1, deg2, b1.reshape(1, _H), W2)
    a2 = _sc_agg()(t2, edges, zeros_e)[:, :_N, :]
    t3 = _mm_mid(a2, t2, deg2, b2.reshape(1, _H), W3)
    a3 = _sc_agg()(t3, edges, zeros_e)[:, :_N, :]
    return _final(a3, t3, deg2, b3.reshape(1, _H))


# speedup vs baseline: 5.3748x; 1.0683x over previous
import functools

import jax
import jax.numpy as jnp
from jax import lax
from jax.experimental import pallas as pl
from jax.experimental.pallas import tpu as pltpu
from jax.experimental.pallas import tpu_sc as plsc

_N = 10000
_E = 160000
_H = 512
_NCH = 4
_CD = 128
_NCORE = 2
_NSUB = 16
_NP = 10240
_RT = _NP // _NSUB
_EP = 163840
_BE = 32
_BD = 128
_ET = _EP // _NSUB
_J = _ET // _BE
_JD = _ET // _BD
_NB = 1000
_G = _N // _NB


def _mesh():
    return plsc.VectorSubcoreMesh(core_axis_name="c", subcore_axis_name="s",
                                  num_cores=_NCORE, num_subcores=_NSUB)


def _sc_deg_body(edges_hbm, zeros_hbm, ones_hbm, out_hbm, cbuf, val, deg_sh):
    core = lax.axis_index("c")
    sub = lax.axis_index("s")
    r0 = sub * _RT
    pltpu.sync_copy(zeros_hbm, deg_sh.at[pl.ds(r0, _RT)])
    pltpu.sync_copy(ones_hbm, val)
    plsc.subcore_barrier()

    def body(j, c):
        g = sub * _JD + j
        pltpu.sync_copy(edges_hbm.at[g], cbuf.at[0])
        pltpu.sync_copy(val, deg_sh.at[cbuf.at[0].at[1]], add=True)
        return c

    lax.fori_loop(0, _JD, body, 0)
    plsc.subcore_barrier()

    @pl.when(core == 0)
    def _():
        pltpu.sync_copy(deg_sh.at[pl.ds(r0, _RT)], out_hbm.at[pl.ds(r0, _RT)])


@functools.lru_cache(maxsize=None)
def _sc_deg():
    return pl.kernel(
        _sc_deg_body,
        out_type=jax.ShapeDtypeStruct((_NP, _CD), jnp.float32),
        mesh=_mesh(),
        scratch_types=[
            pltpu.VMEM((1, 2, _BD), jnp.int32),
            pltpu.VMEM((_BD, _CD), jnp.float32),
            pltpu.VMEM_SHARED((_NP, _CD), jnp.float32),
        ],
    )


_NS = 8
_NG = _J // _NS


def _sc_agg_body(t_hbm, edges_hbm, zeros_hbm, out_hbm, ibuf, rows, acc_sh,
                 *sems):
    core = lax.axis_index("c")
    sub = lax.axis_index("s")
    r0 = sub * _RT
    isems = sems[0:2]
    gsems = sems[2:2 + _NS]
    ssems = sems[2 + _NS:2 + 2 * _NS]
    e0 = sub * _J

    for p in range(2):
        ch = core * 2 + p
        pltpu.sync_copy(zeros_hbm, acc_sh.at[pl.ds(r0, _RT)])
        plsc.subcore_barrier()
        pltpu.async_copy(edges_hbm.at[pl.ds(e0, _NS)], ibuf.at[0], isems[0])

        def sgroup(g2, c):
            for par in range(2):
                g = g2 * 2 + par
                pltpu.make_async_copy(edges_hbm.at[pl.ds(e0 + g * _NS, _NS)],
                                      ibuf.at[par], isems[par]).wait()

                def retire():
                    for k in range(_NS):
                        pltpu.make_async_copy(
                            rows.at[k], acc_sh.at[ibuf.at[par].at[k].at[1]],
                            ssems[k]).wait()

                if par == 0:
                    pl.when(g2 > 0)(retire)
                else:
                    retire()

                @pl.when(g < _NG - 1)
                def _():
                    pltpu.async_copy(
                        edges_hbm.at[pl.ds(e0 + (g + 1) * _NS, _NS)],
                        ibuf.at[1 - par], isems[1 - par])

                for k in range(_NS):
                    pltpu.async_copy(t_hbm.at[ch].at[ibuf.at[par].at[k].at[0]],
                                     rows.at[k], gsems[k])
                for k in range(_NS):
                    pltpu.make_async_copy(
                        t_hbm.at[ch].at[ibuf.at[par].at[k].at[0]],
                        rows.at[k], gsems[k]).wait()
                    pltpu.async_copy(rows.at[k],
                                     acc_sh.at[ibuf.at[par].at[k].at[1]],
                                     ssems[k], add=True)
            return c

        lax.fori_loop(0, _NG // 2, sgroup, 0)
        for k in range(_NS):
            pltpu.make_async_copy(rows.at[k],
                                  acc_sh.at[ibuf.at[1].at[k].at[1]],
                                  ssems[k]).wait()
        plsc.subcore_barrier()
        pltpu.sync_copy(acc_sh.at[pl.ds(r0, _RT)],
                        out_hbm.at[ch].at[pl.ds(r0, _RT)])
        plsc.subcore_barrier()


@functools.lru_cache(maxsize=None)
def _sc_agg():
    return pl.kernel(
        _sc_agg_body,
        out_type=jax.ShapeDtypeStruct((_NCH, _NP, _CD), jnp.float32),
        mesh=_mesh(),
        scratch_types=[
            pltpu.VMEM((2, _NS, 2, _BE), jnp.int32),
            pltpu.VMEM((_NS, _BE, _CD), jnp.float32),
            pltpu.VMEM_SHARED((_NP, _CD), jnp.float32),
        ] + [pltpu.SemaphoreType.DMA] * (2 + 2 * _NS),
    )


def _mm_first_body(x_ref, w_ref, deg_ref, o_ref):
    dis = lax.rsqrt(deg_ref[...] + 1.0)
    y = jnp.dot(x_ref[...], w_ref[...], preferred_element_type=jnp.float32,
                precision=lax.Precision.HIGHEST)
    t = y * dis
    for c in range(_NCH):
        o_ref[c] = t[:, c * _CD:(c + 1) * _CD]


def _mm_first(x, w, deg2):
    return pl.pallas_call(
        _mm_first_body,
        grid=(_G,),
        in_specs=[
            pl.BlockSpec((_NB, x.shape[1]), lambda i: (i, 0)),
            pl.BlockSpec((w.shape[0], _H), lambda i: (0, 0)),
            pl.BlockSpec((_NB, 1), lambda i: (i, 0)),
        ],
        out_specs=pl.BlockSpec((_NCH, _NB, _CD), lambda i: (0, i, 0)),
        out_shape=jax.ShapeDtypeStruct((_NCH, _N, _CD), jnp.float32),
    )(x, w, deg2)


def _mm_mid_body(a_ref, t_ref, deg_ref, b_ref, w_ref, o_ref):
    dis = lax.rsqrt(deg_ref[...] + 1.0)
    s = jnp.concatenate([a_ref[c] + t_ref[c] for c in range(_NCH)], axis=1)
    h = s * dis + b_ref[...]
    h = jnp.where(h > 0, h, jnp.exp(jnp.minimum(h, 0.0)) - 1.0)
    y = jnp.dot(h, w_ref[...], preferred_element_type=jnp.float32,
                precision=lax.Precision.HIGHEST)
    t = y * dis
    for c in range(_NCH):
        o_ref[c] = t[:, c * _CD:(c + 1) * _CD]


def _mm_mid(a, t, deg2, b2d, w):
    return pl.pallas_call(
        _mm_mid_body,
        grid=(_G,),
        in_specs=[
            pl.BlockSpec((_NCH, _NB, _CD), lambda i: (0, i, 0)),
            pl.BlockSpec((_NCH, _NB, _CD), lambda i: (0, i, 0)),
            pl.BlockSpec((_NB, 1), lambda i: (i, 0)),
            pl.BlockSpec((1, _H), lambda i: (0, 0)),
            pl.BlockSpec((_H, _H), lambda i: (0, 0)),
        ],
        out_specs=pl.BlockSpec((_NCH, _NB, _CD), lambda i: (0, i, 0)),
        out_shape=jax.ShapeDtypeStruct((_NCH, _N, _CD), jnp.float32),
    )(a, t, deg2, b2d, w)


def _final_body(a_ref, t_ref, deg_ref, b_ref, o_ref):
    dis = lax.rsqrt(deg_ref[...] + 1.0)
    s = jnp.concatenate([a_ref[c] + t_ref[c] for c in range(_NCH)], axis=1)
    o_ref[...] = s * dis + b_ref[...]


def _final(a, t, deg2, b2d):
    return pl.pallas_call(
        _final_body,
        grid=(_G,),
        in_specs=[
            pl.BlockSpec((_NCH, _NB, _CD), lambda i: (0, i, 0)),
            pl.BlockSpec((_NCH, _NB, _CD), lambda i: (0, i, 0)),
            pl.BlockSpec((_NB, 1), lambda i: (i, 0)),
            pl.BlockSpec((1, _H), lambda i: (0, 0)),
        ],
        out_specs=pl.BlockSpec((_NB, _H), lambda i: (i, 0)),
        out_shape=jax.ShapeDtypeStruct((_N, _H), jnp.float32),
    )(a, t, deg2, b2d)


def kernel(x, edge_index, W1, b1, W2, b2, W3, b3):
    row = edge_index[0].astype(jnp.int32)
    col = edge_index[1].astype(jnp.int32)
    pad = _EP - _E
    rowp = jnp.concatenate([row, jnp.zeros((pad,), jnp.int32)])
    colp = jnp.concatenate([col, jnp.full((pad,), _N, jnp.int32)])
    edges_a = jnp.stack([rowp.reshape(_EP // _BE, _BE),
                         colp.reshape(_EP // _BE, _BE)], axis=1)
    edges_d = jnp.stack([rowp.reshape(_EP // _BD, _BD),
                         colp.reshape(_EP // _BD, _BD)], axis=1)
    zeros_e = jnp.zeros((_RT, _CD), jnp.float32)

    ones_d = jnp.zeros((_BD, _CD), jnp.float32).at[:, 0].set(1.0)
    degt = _sc_deg()(edges_d, zeros_e, ones_d)
    deg2 = degt[:_N, 0:1]

    t1 = _mm_first(x, W1, deg2)
    a1 = _sc_agg()(t1, edges_a, zeros_e)[:, :_N, :]
    t2 = _mm_mid(a1, t1, deg2, b1.reshape(1, _H), W2)
    a2 = _sc_agg()(t2, edges_a, zeros_e)[:, :_N, :]
    t3 = _mm_mid(a2, t2, deg2, b2.reshape(1, _H), W3)
    a3 = _sc_agg()(t3, edges_a, zeros_e)[:, :_N, :]
    return _final(a3, t3, deg2, b3.reshape(1, _H))
